# baseline (speedup 1.0000x reference)
import functools

import jax
import jax.numpy as jnp
from jax import lax
from jax.experimental import pallas as pl
from jax.experimental.pallas import tpu as pltpu
from jax.experimental.pallas import tpu_sc as plsc

_K = 4
_C = _K * 128


@functools.partial(jax.jit, static_argnums=(2, 3, 4))
def _lookup(idx2d, table, n_idx, d, nw_nc):
    nw, nc = nw_nc
    n_per_w = n_idx // nw
    n_chunks = n_per_w // _C
    assert n_chunks % 2 == 0 and n_chunks >= 4

    mesh = plsc.VectorSubcoreMesh(core_axis_name="c", subcore_axis_name="s")

    @functools.partial(
        pl.kernel,
        mesh=mesh,
        out_type=jax.ShapeDtypeStruct((n_idx, d), jnp.float32),
        compiler_params=pltpu.CompilerParams(
            use_tc_tiling_on_sc=False, skip_device_barrier=True
        ),
        scratch_types=[
            pltpu.VMEM((_K, 128), jnp.int32),
            pltpu.VMEM((_K, 128), jnp.int32),
            pltpu.VMEM((_C, d), jnp.float32),
            pltpu.VMEM((_C, d), jnp.float32),
            pltpu.SemaphoreType.DMA,
            pltpu.SemaphoreType.DMA,
            pltpu.SemaphoreType.DMA,
            pltpu.SemaphoreType.DMA,
            pltpu.SemaphoreType.DMA,
        ],
    )
    def body(idx_hbm, table_hbm, out_hbm, idx0, idx1, rows0, rows1,
             sem_l0, sem_l1, sem_w0, sem_w1, sem_g):
        wid = lax.axis_index("s") * nc + lax.axis_index("c")
        base = wid * n_per_w
        base_row = wid * (n_per_w // 128)
        idx_b = (idx0, idx1)
        rows_b = (rows0, rows1)
        sem_l = (sem_l0, sem_l1)
        sem_w = (sem_w0, sem_w1)

        def idx_src(c):
            return idx_hbm.at[pl.ds(base_row + c * _K, _K)]

        def out_dst(c):
            return out_hbm.at[pl.ds(base + c * _C, _C)]

        def fire_gathers(b):
            return [
                pltpu.async_copy(
                    table_hbm.at[idx_b[b].at[j]],
                    rows_b[b].at[pl.ds(j * 128, 128)],
                    sem_g,
                )
                for j in range(_K)
            ]

        for b in range(2):
            pltpu.sync_copy(idx_src(b), idx_b[b])
            for cp in fire_gathers(b):
                cp.wait()
            pltpu.async_copy(idx_src(b + 2), idx_b[b], sem_l[b])
            pltpu.async_copy(rows_b[b], out_dst(b), sem_w[b])

        def steady(i, carry):
            g0 = 2 + 2 * i
            for b in range(2):
                c = g0 + b
                pltpu.make_async_copy(idx_src(c), idx_b[b], sem_l[b]).wait()
                pltpu.make_async_copy(rows_b[b], out_dst(c), sem_w[b]).wait()
                for cp in fire_gathers(b):
                    cp.wait()
                pltpu.async_copy(idx_src(c + 2), idx_b[b], sem_l[b])
                pltpu.async_copy(rows_b[b], out_dst(c), sem_w[b])
            return carry

        lax.fori_loop(0, (n_chunks - 4) // 2, steady, 0, unroll=False)

        for b in range(2):
            c = n_chunks - 2 + b
            pltpu.make_async_copy(idx_src(c), idx_b[b], sem_l[b]).wait()
            pltpu.make_async_copy(rows_b[b], out_dst(c), sem_w[b]).wait()
            for cp in fire_gathers(b):
                cp.wait()
            cp = pltpu.async_copy(rows_b[b], out_dst(c), sem_w[b])
            cp.wait()

    return body(idx2d, table)


def kernel(x, table):
    b, s = x.shape
    v, d = table.shape
    n_idx = b * s
    info = plsc.get_sparse_core_info()
    nc, ns = info.num_cores, info.num_subcores
    nw = nc * ns
    idx2d = x.reshape(n_idx // 128, 128)
    t00 = table[0, 0]
    one = jnp.exp(t00 - t00)
    out = _lookup(idx2d, table * one, n_idx, d, (nw, nc))
    return (out * one).reshape(b, s, d)


# speedup vs baseline: 1.4730x; 1.4730x over previous
import functools

import jax
import jax.numpy as jnp
from jax import lax
from jax.experimental import pallas as pl
from jax.experimental.pallas import tpu as pltpu
from jax.experimental.pallas import tpu_sc as plsc

_NB = 4


@functools.partial(jax.jit, static_argnums=(2,))
def _lookup(x, table, nc):
    b, s = x.shape
    v, d = table.shape
    nw = nc * plsc.get_sparse_core_info().num_subcores
    rows_per_w = b // nw
    n_chunks = rows_per_w // _NB
    assert n_chunks % 2 == 0 and n_chunks >= 4
    segs = []
    off = 0
    while off < s:
        ln = min(128, s - off)
        segs.append((off, ln))
        off += ln

    mesh = plsc.VectorSubcoreMesh(core_axis_name="c", subcore_axis_name="s")

    @functools.partial(
        pl.kernel,
        mesh=mesh,
        out_type=jax.ShapeDtypeStruct((b, s, d), jnp.float32),
        compiler_params=pltpu.CompilerParams(use_tc_tiling_on_sc=False),
        scratch_types=[
            pltpu.VMEM((_NB, s), jnp.int32),
            pltpu.VMEM((_NB, s), jnp.int32),
            pltpu.VMEM((_NB, s, d), jnp.float32),
            pltpu.VMEM((_NB, s, d), jnp.float32),
            pltpu.SemaphoreType.DMA,
            pltpu.SemaphoreType.DMA,
            pltpu.SemaphoreType.DMA,
            pltpu.SemaphoreType.DMA,
            pltpu.SemaphoreType.DMA,
        ],
    )
    def body(x_hbm, table_hbm, out_hbm, idx0, idx1, rows0, rows1,
             sem_l0, sem_l1, sem_w0, sem_w1, sem_g):
        wid = lax.axis_index("s") * nc + lax.axis_index("c")
        base_b = wid * rows_per_w
        idx_b = (idx0, idx1)
        rows_b = (rows0, rows1)
        sem_l = (sem_l0, sem_l1)
        sem_w = (sem_w0, sem_w1)

        def idx_src(c):
            return x_hbm.at[pl.ds(base_b + c * _NB, _NB), :]

        def out_dst(c):
            return out_hbm.at[pl.ds(base_b + c * _NB, _NB)]

        def fire_gathers(bf):
            return [
                pltpu.async_copy(
                    table_hbm.at[idx_b[bf].at[i, pl.ds(o, ln)]],
                    rows_b[bf].at[i, pl.ds(o, ln)],
                    sem_g,
                )
                for i in range(_NB)
                for (o, ln) in segs
            ]

        for bf in range(2):
            pltpu.sync_copy(idx_src(bf), idx_b[bf])
            for cp in fire_gathers(bf):
                cp.wait()
            pltpu.async_copy(idx_src(bf + 2), idx_b[bf], sem_l[bf])
            pltpu.async_copy(rows_b[bf], out_dst(bf), sem_w[bf])

        def steady(it, carry):
            g0 = 2 + 2 * it
            for bf in range(2):
                c = g0 + bf
                pltpu.make_async_copy(idx_src(c), idx_b[bf], sem_l[bf]).wait()
                pltpu.make_async_copy(rows_b[bf], out_dst(c), sem_w[bf]).wait()
                for cp in fire_gathers(bf):
                    cp.wait()
                pltpu.async_copy(idx_src(c + 2), idx_b[bf], sem_l[bf])
                pltpu.async_copy(rows_b[bf], out_dst(c), sem_w[bf])
            return carry

        lax.fori_loop(0, (n_chunks - 4) // 2, steady, 0, unroll=False)

        for bf in range(2):
            c = n_chunks - 2 + bf
            pltpu.make_async_copy(idx_src(c), idx_b[bf], sem_l[bf]).wait()
            pltpu.make_async_copy(rows_b[bf], out_dst(c), sem_w[bf]).wait()
            for cp in fire_gathers(bf):
                cp.wait()
            cp = pltpu.async_copy(rows_b[bf], out_dst(c), sem_w[bf])
            cp.wait()

    return body(x, table)


def kernel(x, table):
    nc = plsc.get_sparse_core_info().num_cores
    return _lookup(x, table, nc)


# speedup vs baseline: 1.4764x; 1.0023x over previous
import functools

import jax
import jax.numpy as jnp
from jax import lax
from jax.experimental import pallas as pl
from jax.experimental.pallas import tpu as pltpu
from jax.experimental.pallas import tpu_sc as plsc

_NB = 4


@functools.partial(jax.jit, static_argnums=(2,))
def _lookup(x, table, nc):
    b, s = x.shape
    v, d = table.shape
    nw = nc * plsc.get_sparse_core_info().num_subcores
    rows_per_w = b // nw
    n_chunks = rows_per_w // _NB
    assert n_chunks % 2 == 0 and n_chunks >= 4
    segs = []
    off = 0
    while off < s:
        ln = min(128, s - off)
        segs.append((off, ln))
        off += ln

    mesh = plsc.VectorSubcoreMesh(core_axis_name="c", subcore_axis_name="s")

    @functools.partial(
        pl.kernel,
        mesh=mesh,
        out_type=jax.ShapeDtypeStruct((b, s, d), jnp.float32),
        compiler_params=pltpu.CompilerParams(
            use_tc_tiling_on_sc=False,
            allow_input_fusion=[True, True],
        ),
        scratch_types=[
            pltpu.VMEM((_NB, s), jnp.int32),
            pltpu.VMEM((_NB, s), jnp.int32),
            pltpu.VMEM((_NB, s, d), jnp.float32),
            pltpu.VMEM((_NB, s, d), jnp.float32),
            pltpu.SemaphoreType.DMA,
            pltpu.SemaphoreType.DMA,
            pltpu.SemaphoreType.DMA,
            pltpu.SemaphoreType.DMA,
            pltpu.SemaphoreType.DMA,
        ],
    )
    def body(x_hbm, table_hbm, out_hbm, idx0, idx1, rows0, rows1,
             sem_l0, sem_l1, sem_w0, sem_w1, sem_g):
        wid = lax.axis_index("s") * nc + lax.axis_index("c")
        base_b = wid * rows_per_w
        idx_b = (idx0, idx1)
        rows_b = (rows0, rows1)
        sem_l = (sem_l0, sem_l1)
        sem_w = (sem_w0, sem_w1)

        def idx_src(c):
            return x_hbm.at[pl.ds(base_b + c * _NB, _NB), :]

        def out_dst(c):
            return out_hbm.at[pl.ds(base_b + c * _NB, _NB)]

        def fire_gathers(bf):
            return [
                pltpu.async_copy(
                    table_hbm.at[idx_b[bf].at[i, pl.ds(o, ln)]],
                    rows_b[bf].at[i, pl.ds(o, ln)],
                    sem_g,
                )
                for i in range(_NB)
                for (o, ln) in segs
            ]

        for bf in range(2):
            pltpu.sync_copy(idx_src(bf), idx_b[bf])
            for cp in fire_gathers(bf):
                cp.wait()
            pltpu.async_copy(idx_src(bf + 2), idx_b[bf], sem_l[bf])
            pltpu.async_copy(rows_b[bf], out_dst(bf), sem_w[bf])

        def steady(it, carry):
            g0 = 2 + 2 * it
            for bf in range(2):
                c = g0 + bf
                pltpu.make_async_copy(idx_src(c), idx_b[bf], sem_l[bf]).wait()
                pltpu.make_async_copy(rows_b[bf], out_dst(c), sem_w[bf]).wait()
                for cp in fire_gathers(bf):
                    cp.wait()
                pltpu.async_copy(idx_src(c + 2), idx_b[bf], sem_l[bf])
                pltpu.async_copy(rows_b[bf], out_dst(c), sem_w[bf])
            return carry

        lax.fori_loop(0, (n_chunks - 4) // 2, steady, 0, unroll=False)

        for bf in range(2):
            c = n_chunks - 2 + bf
            pltpu.make_async_copy(idx_src(c), idx_b[bf], sem_l[bf]).wait()
            pltpu.make_async_copy(rows_b[bf], out_dst(c), sem_w[bf]).wait()
            for cp in fire_gathers(bf):
                cp.wait()
            cp = pltpu.async_copy(rows_b[bf], out_dst(c), sem_w[bf])
            cp.wait()

    return body(x, table)


def kernel(x, table):
    nc = plsc.get_sparse_core_info().num_cores
    return _lookup(x, table, nc)
